# Initial kernel scaffold; baseline (speedup 1.0000x reference)
import jax
import jax.numpy as jnp
from jax.experimental import pallas as pl


def kernel(x, piece_count, table, bias1, W2, b2, Wcp, bcp, Wwdl, bwdl):
    raise NotImplementedError("write your pallas kernel here")


# speedup vs baseline: 1.4371x; 1.4371x over previous
import functools

import jax
import jax.numpy as jnp
from jax import lax
from jax.experimental import pallas as pl
from jax.experimental.pallas import tpu as pltpu
from jax.experimental.pallas import tpu_sc as plsc

B = 16384
K = 32
H = 1024
H2 = 32
NBUCKET = 8
NC = 2
NS = 16
NW = NC * NS
SPW = B // NW
LANES = 16

TC_ROWS = 1024


def _screlu(v):
    return jnp.clip(v, 0.0, 1.0) ** 2


def _sc_body(x_hbm, table_hbm, out_hbm, idx_v, rows_v, acc_v, sem):
    wid = lax.axis_index("c") * NS + lax.axis_index("s")
    base = wid * SPW
    pltpu.sync_copy(x_hbm.at[pl.ds(base, SPW)], idx_v)

    def sample_body(s, carry):
        pltpu.async_copy(table_hbm.at[idx_v.at[s]], rows_v, sem).wait()

        def col_body(c, carry2):
            co = c * LANES
            acc = rows_v[0, pl.ds(co, LANES)]
            for r in range(1, K):
                acc = acc + rows_v[r, pl.ds(co, LANES)]
            acc_v[pl.ds(co, LANES)] = acc
            return carry2

        lax.fori_loop(0, H // LANES, col_body, 0, unroll=False)
        pltpu.sync_copy(acc_v, out_hbm.at[base + s])
        return carry

    lax.fori_loop(0, SPW, sample_body, 0, unroll=False)


def _sc_gather_sum(x, table):
    mesh = plsc.VectorSubcoreMesh(
        core_axis_name="c", subcore_axis_name="s", num_cores=NC, num_subcores=NS
    )
    f = pl.kernel(
        _sc_body,
        out_type=jax.ShapeDtypeStruct((B, H), jnp.float32),
        mesh=mesh,
        scratch_types=[
            pltpu.VMEM((SPW, K), jnp.int32),
            pltpu.VMEM((K, H), jnp.float32),
            pltpu.VMEM((H,), jnp.float32),
            pltpu.SemaphoreType.DMA,
        ],
    )
    return f(x, table)


def _tc_body(h_ref, pc_ref, bias1_ref, w2_ref, b2_ref, wcp_ref, bcp_ref,
             wwdl_ref, bwdl_ref, cp_ref, wdl_ref):
    f32 = jnp.float32
    h = _screlu(h_ref[...] + bias1_ref[...][None, :])
    h2 = _screlu(
        lax.dot_general(h, w2_ref[...], (((1,), (1,)), ((), ())),
                        preferred_element_type=f32)
        + b2_ref[...][None, :]
    )
    cp_all = lax.dot_general(h2, wcp_ref[...], (((1,), (1,)), ((), ())),
                             preferred_element_type=f32) + bcp_ref[...][None, :]
    wdl_all = lax.dot_general(h2, wwdl_ref[...], (((1,), (1,)), ((), ())),
                              preferred_element_type=f32) + bwdl_ref[...][None, :]

    pc = pc_ref[...]
    bucket = jnp.clip(((pc - 2) * NBUCKET) // 30, 0, NBUCKET - 1)
    r = pc.shape[0]
    oh8 = (bucket[:, None]
           == lax.broadcasted_iota(jnp.int32, (r, NBUCKET), 1)).astype(f32)
    cp_ref[...] = jnp.sum(cp_all * oh8, axis=1, keepdims=True)

    oh24 = (bucket[:, None]
            == lax.broadcasted_iota(jnp.int32, (r, 3 * NBUCKET), 1) // 3
            ).astype(f32)
    sel = (lax.broadcasted_iota(jnp.int32, (3 * NBUCKET, 3), 0) % 3
           == lax.broadcasted_iota(jnp.int32, (3 * NBUCKET, 3), 1)).astype(f32)
    wdl_ref[...] = lax.dot_general(wdl_all * oh24, sel,
                                   (((1,), (0,)), ((), ())),
                                   preferred_element_type=f32)


def _tc_mlp(h_pre, piece_count, bias1, w2, b2, wcp, bcp, wwdl, bwdl,
            interpret=False):
    nblk = B // TC_ROWS
    full = lambda shape: pl.BlockSpec(shape, lambda i: tuple(0 for _ in shape))
    return pl.pallas_call(
        _tc_body,
        grid=(nblk,),
        in_specs=[
            pl.BlockSpec((TC_ROWS, H), lambda i: (i, 0)),
            pl.BlockSpec((TC_ROWS,), lambda i: (i,)),
            full((H,)),
            full((H2, H)),
            full((H2,)),
            full((NBUCKET, H2)),
            full((NBUCKET,)),
            full((3 * NBUCKET, H2)),
            full((3 * NBUCKET,)),
        ],
        out_specs=[
            pl.BlockSpec((TC_ROWS, 1), lambda i: (i, 0)),
            pl.BlockSpec((TC_ROWS, 3), lambda i: (i, 0)),
        ],
        out_shape=[
            jax.ShapeDtypeStruct((B, 1), jnp.float32),
            jax.ShapeDtypeStruct((B, 3), jnp.float32),
        ],
        interpret=interpret,
    )(h_pre, piece_count, bias1, w2, b2, wcp, bcp, wwdl, bwdl)


def kernel(x, piece_count, table, bias1, W2, b2, Wcp, bcp, Wwdl, bwdl):
    h_pre = _sc_gather_sum(x, table)
    cp, wdl = _tc_mlp(h_pre, piece_count, bias1, W2, b2, Wcp, bcp, Wwdl, bwdl)
    return (cp, wdl)
